# baseline (speedup 1.0000x reference)
import jax
import jax.numpy as jnp
from jax.experimental import pallas as pl
from jax.experimental.pallas import tpu as pltpu

EPS = 1e-10
B = 128
V = 100000
BV = 2048
NV = (V + BV - 1) // BV
BLK2 = 800
NB2 = V // BLK2


def _argmax_body(logits_ref, u_ref, zeros_ref, idx_ref, m_ref):
    i = pl.program_id(0)
    z = logits_ref[...] - jnp.log(-jnp.log(u_ref[...] + EPS) + EPS)
    gcol = jax.lax.broadcasted_iota(jnp.int32, z.shape, 1) + i * BV
    z = jnp.where(gcol < V, z, -jnp.inf)
    bmax = jnp.max(z, axis=1, keepdims=True)
    bidx = jnp.min(jnp.where(z == bmax, gcol, V), axis=1, keepdims=True)
    zeros_ref[...] = jnp.zeros_like(zeros_ref)

    @pl.when(i == 0)
    def _():
        m_ref[...] = bmax
        idx_ref[...] = bidx

    @pl.when(i != 0)
    def _():
        better = bmax > m_ref[...]
        m_ref[...] = jnp.where(better, bmax, m_ref[...])
        idx_ref[...] = jnp.where(better, bidx, idx_ref[...])


def _scatter_body(idx_ref, zin_ref, out_ref):
    del zin_ref
    j = idx_ref[pl.program_id(0)]
    col = jax.lax.broadcasted_iota(jnp.int32, (1, 1, 1, BLK2), 3)
    out_ref[...] = (col == j % BLK2).astype(jnp.float32)


def kernel(logits, u):
    zeros, idx = pl.pallas_call(
        _argmax_body,
        grid=(NV,),
        in_specs=[
            pl.BlockSpec((B, BV), lambda i: (0, i)),
            pl.BlockSpec((B, BV), lambda i: (0, i)),
        ],
        out_specs=[
            pl.BlockSpec((B, BV), lambda i: (0, i)),
            pl.BlockSpec((B, 1), lambda i: (0, 0)),
        ],
        out_shape=[
            jax.ShapeDtypeStruct((B, V), jnp.float32),
            jax.ShapeDtypeStruct((B, 1), jnp.int32),
        ],
        scratch_shapes=[pltpu.VMEM((B, 1), jnp.float32)],
    )(logits, u)

    out = pl.pallas_call(
        _scatter_body,
        grid_spec=pltpu.PrefetchScalarGridSpec(
            num_scalar_prefetch=1,
            grid=(B,),
            in_specs=[pl.BlockSpec((1, 1, 1, BLK2),
                                   lambda r, idx: (r, idx[r] // BLK2, 0, 0))],
            out_specs=pl.BlockSpec((1, 1, 1, BLK2),
                                   lambda r, idx: (r, idx[r] // BLK2, 0, 0)),
        ),
        out_shape=jax.ShapeDtypeStruct((B, NB2, 1, BLK2), jnp.float32),
        input_output_aliases={1: 0},
    )(idx.reshape(B), zeros.reshape(B, NB2, 1, BLK2))
    return out.reshape(B, V)


# speedup vs baseline: 2.4013x; 2.4013x over previous
import jax
import jax.numpy as jnp
from jax.experimental import pallas as pl
from jax.experimental.pallas import tpu as pltpu

EPS = 1e-10
B = 128
V = 100000
BV = 2048
NV = (V + BV - 1) // BV
BR = 8
NR = B // BR


def _argmax_body(logits_ref, u_ref, idx_ref, m_ref):
    i = pl.program_id(0)
    z = logits_ref[...] - jnp.log(-jnp.log(u_ref[...] + EPS) + EPS)
    gcol = jax.lax.broadcasted_iota(jnp.int32, z.shape, 1) + i * BV
    z = jnp.where(gcol < V, z, -jnp.inf)
    bmax = jnp.max(z, axis=1, keepdims=True)
    bidx = jnp.min(jnp.where(z == bmax, gcol, V), axis=1, keepdims=True)

    @pl.when(i == 0)
    def _():
        m_ref[...] = bmax
        idx_ref[...] = bidx

    @pl.when(i != 0)
    def _():
        better = bmax > m_ref[...]
        m_ref[...] = jnp.where(better, bmax, m_ref[...])
        idx_ref[...] = jnp.where(better, bidx, idx_ref[...])


def _onehot_body(idx_ref, out_ref):
    col = jax.lax.broadcasted_iota(jnp.int32, (BR, V), 1)
    out_ref[...] = (col == idx_ref[...]).astype(jnp.float32)


def kernel(logits, u):
    idx = pl.pallas_call(
        _argmax_body,
        grid=(NV,),
        in_specs=[
            pl.BlockSpec((B, BV), lambda i: (0, i)),
            pl.BlockSpec((B, BV), lambda i: (0, i)),
        ],
        out_specs=pl.BlockSpec((B, 1), lambda i: (0, 0)),
        out_shape=jax.ShapeDtypeStruct((B, 1), jnp.int32),
        scratch_shapes=[pltpu.VMEM((B, 1), jnp.float32)],
    )(logits, u)

    out = pl.pallas_call(
        _onehot_body,
        grid=(NR,),
        in_specs=[pl.BlockSpec((BR, 1), lambda i: (i, 0))],
        out_specs=pl.BlockSpec((BR, V), lambda i: (i, 0)),
        out_shape=jax.ShapeDtypeStruct((B, V), jnp.float32),
    )(idx)
    return out


# speedup vs baseline: 2.6714x; 1.1125x over previous
import jax
import jax.numpy as jnp
from jax.experimental import pallas as pl

EPS = 1e-10
B = 128
V = 100000
BR = 8
NR = B // BR


def _body(logits_ref, u_ref, out_ref):
    z = logits_ref[...] - jnp.log(-jnp.log(u_ref[...] + EPS) + EPS)
    col = jax.lax.broadcasted_iota(jnp.int32, z.shape, 1)
    bmax = jnp.max(z, axis=1, keepdims=True)
    bidx = jnp.min(jnp.where(z == bmax, col, V), axis=1, keepdims=True)
    out_ref[...] = (col == bidx).astype(jnp.float32)


def kernel(logits, u):
    return pl.pallas_call(
        _body,
        grid=(NR,),
        in_specs=[
            pl.BlockSpec((BR, V), lambda i: (i, 0)),
            pl.BlockSpec((BR, V), lambda i: (i, 0)),
        ],
        out_specs=pl.BlockSpec((BR, V), lambda i: (i, 0)),
        out_shape=jax.ShapeDtypeStruct((B, V), jnp.float32),
    )(logits, u)


# speedup vs baseline: 7.5434x; 2.8238x over previous
import jax
import jax.numpy as jnp
from jax.experimental import pallas as pl
from jax.experimental.pallas import tpu as pltpu

EPS = 1e-10
B = 128
V = 100000
BV = 4096
NV = (V + BV - 1) // BV


def _body(lt_ref, ut_ref, out_ref, m_ref, idx_ref):
    i = pl.program_id(0)

    @pl.when(i < NV)
    def _reduce():
        z = lt_ref[...] - jnp.log(-jnp.log(ut_ref[...] + EPS) + EPS)
        row = jax.lax.broadcasted_iota(jnp.int32, z.shape, 0) + i * BV
        z = jnp.where(row < V, z, -jnp.inf)
        bmax = jnp.max(z, axis=0, keepdims=True)
        bidx = jnp.min(jnp.where(z == bmax, row, V), axis=0, keepdims=True)

        @pl.when(i == 0)
        def _():
            m_ref[...] = bmax
            idx_ref[...] = bidx

        @pl.when(i != 0)
        def _():
            better = bmax > m_ref[...]
            m_ref[...] = jnp.where(better, bmax, m_ref[...])
            idx_ref[...] = jnp.where(better, bidx, idx_ref[...])

    @pl.when(i >= NV)
    def _write():
        row = jax.lax.broadcasted_iota(jnp.int32, (BV, B), 0) + (i - NV) * BV
        out_ref[...] = (row == idx_ref[...]).astype(jnp.float32)


def kernel(logits, u):
    out_t = pl.pallas_call(
        _body,
        grid=(2 * NV,),
        in_specs=[
            pl.BlockSpec((BV, B), lambda i: (jnp.minimum(i, NV - 1), 0)),
            pl.BlockSpec((BV, B), lambda i: (jnp.minimum(i, NV - 1), 0)),
        ],
        out_specs=pl.BlockSpec((BV, B), lambda i: (jnp.maximum(i - NV, 0), 0)),
        out_shape=jax.ShapeDtypeStruct((V, B), jnp.float32),
        scratch_shapes=[
            pltpu.VMEM((1, B), jnp.float32),
            pltpu.VMEM((1, B), jnp.int32),
        ],
    )(logits.T, u.T)
    return out_t.T


# speedup vs baseline: 8.2373x; 1.0920x over previous
import jax
import jax.numpy as jnp
from jax.experimental import pallas as pl
from jax.experimental.pallas import tpu as pltpu

EPS = 1e-10
B = 128
V = 100000
BV = 8192
NV = (V + BV - 1) // BV


def _body(lt_ref, ut_ref, out_ref, m_ref, idx_ref):
    i = pl.program_id(0)

    @pl.when(i < NV)
    def _reduce():
        z = lt_ref[...] - jnp.log(-jnp.log(ut_ref[...] + EPS) + EPS)
        row = jax.lax.broadcasted_iota(jnp.int32, z.shape, 0) + i * BV
        z = jnp.where(row < V, z, -jnp.inf)
        bmax = jnp.max(z, axis=0, keepdims=True)
        bidx = jnp.min(jnp.where(z == bmax, row, V), axis=0, keepdims=True)

        @pl.when(i == 0)
        def _():
            m_ref[...] = bmax
            idx_ref[...] = bidx

        @pl.when(i != 0)
        def _():
            better = bmax > m_ref[...]
            m_ref[...] = jnp.where(better, bmax, m_ref[...])
            idx_ref[...] = jnp.where(better, bidx, idx_ref[...])

    @pl.when(i >= NV)
    def _write():
        row = jax.lax.broadcasted_iota(jnp.int32, (BV, B), 0) + (i - NV) * BV
        out_ref[...] = (row == idx_ref[...]).astype(jnp.float32)


def kernel(logits, u):
    out_t = pl.pallas_call(
        _body,
        grid=(2 * NV,),
        in_specs=[
            pl.BlockSpec((BV, B), lambda i: (jnp.minimum(i, NV - 1), 0)),
            pl.BlockSpec((BV, B), lambda i: (jnp.minimum(i, NV - 1), 0)),
        ],
        out_specs=pl.BlockSpec((BV, B), lambda i: (jnp.maximum(i - NV, 0), 0)),
        out_shape=jax.ShapeDtypeStruct((V, B), jnp.float32),
        scratch_shapes=[
            pltpu.VMEM((1, B), jnp.float32),
            pltpu.VMEM((1, B), jnp.int32),
        ],
    )(logits.T, u.T)
    return out_t.T
